# baseline (speedup 1.0000x reference)
import functools

import jax
import jax.numpy as jnp
import numpy as np
from jax import lax
from jax.experimental import pallas as pl
from jax.experimental.pallas import tpu as pltpu
from jax.experimental.pallas import tpu_sc as plsc

_ALPHA = 4


@functools.lru_cache(maxsize=None)
def _slow_indices(T: int) -> tuple:
    n = T // _ALPHA
    i = np.arange(n - 1, dtype=np.float32) / np.float32(n - 1)
    lo, hi = np.float32(0.0), np.float32(T - 1)
    vals = np.concatenate([lo * (np.float32(1.0) - i) + hi * i, [hi]])
    return tuple(int(v) for v in vals.astype(np.int32))


def _copy_body(in_ref, out_ref):
    out_ref[...] = in_ref[...]


def _fast_copy(frames):
    C, T, H, W = frames.shape
    flat = frames.reshape(C * T, H * W)
    BT = 32
    out = pl.pallas_call(
        _copy_body,
        grid=(C * T // BT,),
        in_specs=[pl.BlockSpec((BT, H * W), lambda i: (i, 0))],
        out_specs=pl.BlockSpec((BT, H * W), lambda i: (i, 0)),
        out_shape=jax.ShapeDtypeStruct((C * T, H * W), frames.dtype),
        compiler_params=pltpu.CompilerParams(
            dimension_semantics=("arbitrary",)
        ),
    )(flat)
    return out.reshape(C, T, H, W)


def _slow_gather_sc(frames, sel):
    C, T, H, W = frames.shape
    S = len(sel)
    FRAME = H * W
    flat = frames.reshape(C * T * FRAME)

    info = plsc.get_sparse_core_info()
    NW = info.num_cores * info.num_subcores
    n_chunks = 96
    per_w = n_chunks // NW
    CH = FRAME // 2

    mesh = plsc.VectorSubcoreMesh(core_axis_name="c", subcore_axis_name="s")

    def src_word(chunk):
        r = chunk // 2
        half = chunk % 2
        ch = r // S
        k = r % S
        src_t = functools.reduce(
            lambda acc, i: jnp.where(k == i, sel[i], acc),
            range(S),
            jnp.int32(0),
        )
        return (ch * T + src_t) * FRAME + half * CH

    @functools.partial(
        pl.kernel,
        out_type=jax.ShapeDtypeStruct((C * S * FRAME,), frames.dtype),
        mesh=mesh,
        scratch_types=[
            pltpu.VMEM((CH,), frames.dtype),
            pltpu.VMEM((CH,), frames.dtype),
            pltpu.SemaphoreType.DMA,
            pltpu.SemaphoreType.DMA,
        ],
    )
    def gather(frames_hbm, slow_hbm, buf0, buf1, sem0, sem1):
        wid = lax.axis_index("s") * info.num_cores + lax.axis_index("c")
        bufs = (buf0, buf1)
        sems = (sem0, sem1)
        reads = [None, None]
        writes = [None, None]
        for j in range(per_w):
            chunk = wid * per_w + j
            b = j % 2
            if writes[b] is not None:
                writes[b].wait()
            reads[b] = pltpu.async_copy(
                frames_hbm.at[pl.ds(src_word(chunk) * 1, CH)], bufs[b], sems[b]
            )
            if j >= 1:
                pb = (j - 1) % 2
                reads[pb].wait()
                writes[pb] = pltpu.async_copy(
                    bufs[pb], slow_hbm.at[pl.ds((chunk - 1) * CH, CH)], sems[pb]
                )
        last = (per_w - 1) % 2
        reads[last].wait()
        writes[last] = pltpu.async_copy(
            bufs[last],
            slow_hbm.at[pl.ds((wid * per_w + per_w - 1) * CH, CH)],
            sems[last],
        )
        for wr in writes:
            if wr is not None:
                wr.wait()

    return gather(flat).reshape(C, S, H, W)


def kernel(frames):
    T = frames.shape[1]
    sel = _slow_indices(T)
    slow = _slow_gather_sc(frames, sel)
    fast = _fast_copy(frames)
    return (slow, fast)


# speedup vs baseline: 3.1437x; 3.1437x over previous
import functools

import jax
import jax.numpy as jnp
import numpy as np
from jax import lax
from jax.experimental import pallas as pl
from jax.experimental.pallas import tpu as pltpu
from jax.experimental.pallas import tpu_sc as plsc

_ALPHA = 4


@functools.lru_cache(maxsize=None)
def _slow_indices(T: int) -> tuple:
    n = T // _ALPHA
    i = np.arange(n - 1, dtype=np.float32) / np.float32(n - 1)
    lo, hi = np.float32(0.0), np.float32(T - 1)
    vals = np.concatenate([lo * (np.float32(1.0) - i) + hi * i, [hi]])
    return tuple(int(v) for v in vals.astype(np.int32))


def _copy_body(in_ref, out_ref):
    out_ref[...] = in_ref[...]


def _fast_copy(frames):
    C, T, H, W = frames.shape
    BT = 16
    return pl.pallas_call(
        _copy_body,
        grid=(C, T // BT),
        in_specs=[pl.BlockSpec((1, BT, H, W), lambda c, i: (c, i, 0, 0))],
        out_specs=pl.BlockSpec((1, BT, H, W), lambda c, i: (c, i, 0, 0)),
        out_shape=jax.ShapeDtypeStruct((C, T, H, W), frames.dtype),
        compiler_params=pltpu.CompilerParams(
            dimension_semantics=("arbitrary", "arbitrary")
        ),
    )(frames)


def _slow_gather_sc(frames, sel):
    C, T, H, W = frames.shape
    S = len(sel)
    HH = H // 2

    info = plsc.get_sparse_core_info()
    NW = info.num_cores * info.num_subcores
    n_chunks = C * S * 2
    per_w = n_chunks // NW

    mesh = plsc.VectorSubcoreMesh(core_axis_name="c", subcore_axis_name="s")

    def chunk_coords(chunk):
        r = chunk // 2
        half = chunk % 2
        ch = r // S
        k = r % S
        src_t = functools.reduce(
            lambda acc, i: jnp.where(k == i, sel[i], acc),
            range(S),
            jnp.int32(0),
        )
        return ch, k, src_t, half * HH

    @functools.partial(
        pl.kernel,
        out_type=jax.ShapeDtypeStruct((C, S, H, W), frames.dtype),
        mesh=mesh,
        scratch_types=[
            pltpu.VMEM((HH, W), frames.dtype),
            pltpu.VMEM((HH, W), frames.dtype),
            pltpu.SemaphoreType.DMA,
            pltpu.SemaphoreType.DMA,
        ],
    )
    def gather(frames_hbm, slow_hbm, buf0, buf1, sem0, sem1):
        wid = lax.axis_index("s") * info.num_cores + lax.axis_index("c")
        bufs = (buf0, buf1)
        sems = (sem0, sem1)
        reads = [None, None]
        writes = [None, None]
        coords = [chunk_coords(wid * per_w + j) for j in range(per_w)]
        for j in range(per_w):
            ch, k, src_t, h0 = coords[j]
            b = j % 2
            if writes[b] is not None:
                writes[b].wait()
            reads[b] = pltpu.async_copy(
                frames_hbm.at[ch, src_t, pl.ds(h0, HH)], bufs[b], sems[b]
            )
            if j >= 1:
                pb = (j - 1) % 2
                pch, pk, _, ph0 = coords[j - 1]
                reads[pb].wait()
                writes[pb] = pltpu.async_copy(
                    bufs[pb], slow_hbm.at[pch, pk, pl.ds(ph0, HH)], sems[pb]
                )
        last = (per_w - 1) % 2
        lch, lk, _, lh0 = coords[per_w - 1]
        reads[last].wait()
        writes[last] = pltpu.async_copy(
            bufs[last], slow_hbm.at[lch, lk, pl.ds(lh0, HH)], sems[last]
        )
        for wr in writes:
            if wr is not None:
                wr.wait()

    return gather(frames)


def kernel(frames):
    T = frames.shape[1]
    sel = _slow_indices(T)
    slow = _slow_gather_sc(frames, sel)
    fast = _fast_copy(frames)
    return (slow, fast)


# speedup vs baseline: 3.2199x; 1.0242x over previous
import functools

import jax
import jax.numpy as jnp
import numpy as np
from jax import lax
from jax.experimental import pallas as pl
from jax.experimental.pallas import tpu as pltpu
from jax.experimental.pallas import tpu_sc as plsc

_ALPHA = 4


@functools.lru_cache(maxsize=None)
def _slow_indices(T: int) -> tuple:
    n = T // _ALPHA
    i = np.arange(n - 1, dtype=np.float32) / np.float32(n - 1)
    lo, hi = np.float32(0.0), np.float32(T - 1)
    vals = np.concatenate([lo * (np.float32(1.0) - i) + hi * i, [hi]])
    return tuple(int(v) for v in vals.astype(np.int32))


def _copy_body(in_ref, out_ref):
    out_ref[...] = in_ref[...]


def _fast_copy(frames):
    C, T, H, W = frames.shape
    BT = 32
    return pl.pallas_call(
        _copy_body,
        grid=(C, T // BT),
        in_specs=[pl.BlockSpec((1, BT, H, W), lambda c, i: (c, i, 0, 0))],
        out_specs=pl.BlockSpec((1, BT, H, W), lambda c, i: (c, i, 0, 0)),
        out_shape=jax.ShapeDtypeStruct((C, T, H, W), frames.dtype),
        compiler_params=pltpu.CompilerParams(
            dimension_semantics=("arbitrary", "arbitrary")
        ),
    )(frames)


def _slow_gather_sc(frames, sel):
    C, T, H, W = frames.shape
    S = len(sel)
    HH = H // 2

    info = plsc.get_sparse_core_info()
    NW = info.num_cores * info.num_subcores
    n_chunks = C * S * 2
    per_w = n_chunks // NW

    mesh = plsc.VectorSubcoreMesh(core_axis_name="c", subcore_axis_name="s")

    def chunk_coords(chunk):
        r = chunk // 2
        half = chunk % 2
        ch = r // S
        k = r % S
        src_t = functools.reduce(
            lambda acc, i: jnp.where(k == i, sel[i], acc),
            range(S),
            jnp.int32(0),
        )
        return ch, k, src_t, half * HH

    @functools.partial(
        pl.kernel,
        out_type=jax.ShapeDtypeStruct((C, S, H, W), frames.dtype),
        mesh=mesh,
        scratch_types=[
            pltpu.VMEM((HH, W), frames.dtype),
            pltpu.VMEM((HH, W), frames.dtype),
            pltpu.VMEM((HH, W), frames.dtype),
            pltpu.SemaphoreType.DMA,
            pltpu.SemaphoreType.DMA,
            pltpu.SemaphoreType.DMA,
        ],
    )
    def gather(frames_hbm, slow_hbm, buf0, buf1, buf2, sem0, sem1, sem2):
        wid = lax.axis_index("s") * info.num_cores + lax.axis_index("c")
        bufs = (buf0, buf1, buf2)
        sems = (sem0, sem1, sem2)
        coords = [chunk_coords(wid * per_w + j) for j in range(per_w)]
        reads = [
            pltpu.async_copy(
                frames_hbm.at[c_, t_, pl.ds(h0, HH)], bufs[j], sems[j]
            )
            for j, (c_, _, t_, h0) in enumerate(coords)
        ]
        writes = []
        for j, (c_, k_, _, h0) in enumerate(coords):
            reads[j].wait()
            writes.append(
                pltpu.async_copy(
                    bufs[j], slow_hbm.at[c_, k_, pl.ds(h0, HH)], sems[j]
                )
            )
        for wr in writes:
            wr.wait()

    return gather(frames)


def kernel(frames):
    T = frames.shape[1]
    sel = _slow_indices(T)
    slow = _slow_gather_sc(frames, sel)
    fast = _fast_copy(frames)
    return (slow, fast)
